# baseline (speedup 1.0000x reference)
import functools

import numpy as np

import jax
import jax.numpy as jnp
from jax.experimental import pallas as pl
from jax.experimental.pallas import tpu as pltpu

SB = 12
CPAD = 16

def _conv1_taps(nbr):
    taps = []
    for ph in range(2):
        for pw in range(2):
            for kappa in range(2):
                hp = (ph + kappa) % 2
                delta = (ph + kappa) // 2
                for dj in ((0,) if pw == 0 else (0, 1)):
                    taps.append((ph * 2 + pw, hp, dj * nbr + delta, kappa, dj))
    return taps


def _fused_kernel(x_ref, w1_ref, b1_ref, w2_ref, b2_ref, w3_ref, b3_ref,
                  wh_ref, bh_ref, wq_ref, bq_ref, o_ref, *, nb):
    nbr = nb * SB
    f32 = jnp.float32

    xcat = []
    for hp in range(2):
        parts = []
        for wp in range(12):
            parts.append(jnp.concatenate(
                [x_ref[0, hp * 4 + dh, :, 32 * wp:32 * (wp + 1)]
                 for dh in range(4)], axis=1))
        parts.append(jnp.zeros((CPAD, 128), f32))
        xcat.append(jnp.concatenate(parts, axis=0))

    n1 = 11 * nbr
    b1 = b1_ref[...]
    accs = [None, None, None, None]
    for t, (ocls, hp, shift, _, _) in enumerate(_conv1_taps(nbr)):
        d = jnp.dot(xcat[hp][shift:shift + n1, :], w1_ref[t],
                    preferred_element_type=f32)
        accs[ocls] = d if accs[ocls] is None else accs[ocls] + d
    zpad1 = jnp.zeros((CPAD, 32), f32)
    y1_parts = []
    for a in accs:
        y1_parts.append(jnp.maximum(a + b1, 0.0))
        y1_parts.append(zpad1)
    y1 = jnp.concatenate(y1_parts, axis=0)
    cstride = n1 + CPAD

    n2 = 9 * nbr
    w2 = w2_ref[...]
    acc2 = None
    for kh in range(4):
        for kw in range(4):
            ph, a = kh % 2, kh // 2
            pw, b_ = kw % 2, kw // 2
            start = (ph * 2 + pw) * cstride + b_ * nbr + a
            tap = kh * 4 + kw
            d = jnp.dot(y1[start:start + n2, :], w2[tap * 32:(tap + 1) * 32, :],
                        preferred_element_type=f32)
            acc2 = d if acc2 is None else acc2 + d
    y2 = jnp.maximum(acc2 + b2_ref[...], 0.0)
    y2 = jnp.concatenate([y2, jnp.zeros((CPAD, 64), f32)], axis=0)

    n3 = 7 * nbr
    w3 = w3_ref[...]
    acc3 = None
    for kh in range(3):
        for kw in range(3):
            start = kw * nbr + kh
            tap = kh * 3 + kw
            d = jnp.dot(y2[start:start + n3, :], w3[tap * 64:(tap + 1) * 64, :],
                        preferred_element_type=f32)
            acc3 = d if acc3 is None else acc3 + d
    y3 = jnp.maximum(acc3 + b3_ref[...], 0.0)

    pieces = [y3[t * nbr:(t + 1) * nbr].reshape(nb, SB, 64)[:, :7, :]
              for t in range(7)]
    feat = jnp.concatenate(pieces, axis=1).reshape(nb, 49 * 64)
    feat = jnp.concatenate([feat, jnp.zeros((nb, 64), f32)], axis=1)

    h = jnp.maximum(
        jnp.dot(feat.astype(jnp.bfloat16), wh_ref[...],
                preferred_element_type=f32) + bh_ref[...], 0.0)
    q = jnp.dot(h, wq_ref[...], preferred_element_type=f32) + bq_ref[...]
    o_ref[0] = q


def kernel(x_nchw, conv1_w, conv1_b, conv2_w, conv2_b, conv3_w, conv3_b,
           sel, wh, bh, wq, bq):
    B = x_nchw.shape[0]
    C = x_nchw.shape[1]
    A = wq.shape[1]
    nb = B // 2

    x = jnp.transpose(x_nchw, (0, 2, 3, 1)).astype(jnp.float32)
    x = jnp.pad(x, ((0, 0), (0, 12), (0, 6), (0, 0)))
    x = x.reshape(2, nb, 12, 2, 4, 12 * 8 * C)
    x = x.transpose(0, 3, 4, 1, 2, 5)
    x = x.reshape(2, 8, nb * SB, 12 * 8 * C)

    taps = _conv1_taps(nb * SB)
    idx = np.zeros((len(taps), 32 * C), np.int32)
    msk = np.zeros((len(taps), 32 * C, 1), np.float32)
    for t, (ocls, _, _, kappa, dj) in enumerate(taps):
        pw = ocls % 2
        for dh in range(4):
            kh = 4 * kappa + dh
            for w8 in range(8):
                kw = w8 + 8 * dj - 4 * pw
                if 0 <= kw < 8:
                    for c in range(C):
                        idx[t, (dh * 8 + w8) * C + c] = (kh * 8 + kw) * C + c
                        msk[t, (dh * 8 + w8) * C + c, 0] = 1.0
    w1t = conv1_w[jnp.asarray(idx.reshape(-1))].reshape(
        len(taps), 32 * C, 32) * jnp.asarray(msk)

    whp = jnp.concatenate(
        [wh[:3136].reshape(7, 7, 64, wh.shape[1]).transpose(1, 0, 2, 3)
         .reshape(3136, wh.shape[1]), wh[3136:]], axis=0)

    args = (x, w1t, conv1_b, conv2_w, conv2_b, conv3_w, conv3_b,
            whp, bh, wq, bq)
    in_specs = [
        pl.BlockSpec((1, 8, nb * SB, 12 * 8 * C), lambda i: (i, 0, 0, 0)),
        pl.BlockSpec(w1t.shape, lambda i: (0, 0, 0)),
    ] + [pl.BlockSpec(a.shape, lambda i: (0,) * a.ndim) for a in args[2:]]

    out = pl.pallas_call(
        functools.partial(_fused_kernel, nb=nb),
        out_shape=jax.ShapeDtypeStruct((2, nb, A), jnp.float32),
        grid=(2,),
        in_specs=in_specs,
        out_specs=pl.BlockSpec((1, nb, A), lambda i: (i, 0, 0)),
        compiler_params=pltpu.CompilerParams(
            dimension_semantics=("parallel",)),
    )(*args)
    return out.reshape(B, A)


# speedup vs baseline: 1.0492x; 1.0492x over previous
import functools

import numpy as np

import jax
import jax.numpy as jnp
from jax.experimental import pallas as pl
from jax.experimental.pallas import tpu as pltpu

SB = 12
CPAD = 16

def _conv1_taps(nbr):
    taps = []
    for ph in range(2):
        for pw in range(2):
            for kappa in range(2):
                hp = (ph + kappa) % 2
                delta = (ph + kappa) // 2
                for dj in ((0,) if pw == 0 else (0, 1)):
                    taps.append((ph * 2 + pw, hp, dj * nbr + delta, kappa, dj))
    return taps


def _fused_kernel(x_ref, w1_ref, b1_ref, w2_ref, b2_ref, w3_ref, b3_ref,
                  wh_ref, bh_ref, wq_ref, bq_ref, o_ref, *, nb):
    nbr = nb * SB
    f32 = jnp.float32

    xcat = []
    for hp in range(2):
        parts = []
        for wp in range(12):
            parts.append(jnp.concatenate(
                [x_ref[0, hp * 4 + dh, :, 32 * wp:32 * (wp + 1)]
                 for dh in range(4)], axis=1))
        parts.append(jnp.zeros((CPAD, 128), f32))
        xcat.append(jnp.concatenate(parts, axis=0))

    n1 = 11 * nbr
    b1 = b1_ref[...]
    accs = [None, None, None, None]
    for t, (ocls, hp, shift, _, _) in enumerate(_conv1_taps(nbr)):
        d = jnp.dot(xcat[hp][shift:shift + n1, :], w1_ref[t],
                    preferred_element_type=f32)
        accs[ocls] = d if accs[ocls] is None else accs[ocls] + d
    zpad1 = jnp.zeros((CPAD, 32), f32)
    y1_parts = []
    for a in accs:
        y1_parts.append(jnp.maximum(a + b1, 0.0))
        y1_parts.append(zpad1)
    y1 = jnp.concatenate(y1_parts, axis=0)
    cstride = n1 + CPAD

    n2 = 9 * nbr
    w2 = w2_ref[...]
    acc2 = None
    for kh in range(4):
        for kw in range(4):
            ph, a = kh % 2, kh // 2
            pw, b_ = kw % 2, kw // 2
            start = (ph * 2 + pw) * cstride + b_ * nbr + a
            tap = kh * 4 + kw
            d = jnp.dot(y1[start:start + n2, :], w2[tap * 32:(tap + 1) * 32, :],
                        preferred_element_type=f32)
            acc2 = d if acc2 is None else acc2 + d
    y2 = jnp.maximum(acc2 + b2_ref[...], 0.0)
    y2 = jnp.concatenate([y2, jnp.zeros((CPAD, 64), f32)], axis=0)

    n3 = 7 * nbr
    w3 = w3_ref[...]
    acc3 = None
    for kh in range(3):
        for kw in range(3):
            start = kw * nbr + kh
            tap = kh * 3 + kw
            d = jnp.dot(y2[start:start + n3, :], w3[tap * 64:(tap + 1) * 64, :],
                        preferred_element_type=f32)
            acc3 = d if acc3 is None else acc3 + d
    y3 = jnp.maximum(acc3 + b3_ref[...], 0.0)

    pieces = [y3[t * nbr:(t + 1) * nbr].reshape(nb, SB, 64)[:, :7, :]
              for t in range(7)]
    feat = jnp.concatenate(pieces, axis=1).reshape(nb, 49 * 64)
    feat = jnp.concatenate([feat, jnp.zeros((nb, 64), f32)], axis=1)

    h = jnp.maximum(
        jnp.dot(feat.astype(jnp.bfloat16), wh_ref[...],
                preferred_element_type=f32) + bh_ref[...], 0.0)
    q = jnp.dot(h, wq_ref[...], preferred_element_type=f32) + bq_ref[...]
    o_ref[0] = q


def kernel(x_nchw, conv1_w, conv1_b, conv2_w, conv2_b, conv3_w, conv3_b,
           sel, wh, bh, wq, bq):
    B = x_nchw.shape[0]
    C = x_nchw.shape[1]
    A = wq.shape[1]
    nb = B // 2

    x = jnp.transpose(x_nchw, (0, 2, 3, 1)).astype(jnp.float32)
    x = jnp.pad(x, ((0, 0), (0, 12), (0, 6), (0, 0)))
    x = x.reshape(2, nb, 12, 2, 4, 12 * 8 * C)
    x = x.transpose(0, 3, 4, 1, 2, 5)
    x = x.reshape(2, 8, nb * SB, 12 * 8 * C)

    taps = _conv1_taps(nb * SB)
    idx = np.zeros((len(taps), 32 * C), np.int32)
    msk = np.zeros((len(taps), 32 * C, 1), np.float32)
    for t, (ocls, _, _, kappa, dj) in enumerate(taps):
        pw = ocls % 2
        for dh in range(4):
            kh = 4 * kappa + dh
            for w8 in range(8):
                kw = w8 + 8 * dj - 4 * pw
                if 0 <= kw < 8:
                    for c in range(C):
                        idx[t, (dh * 8 + w8) * C + c] = (kh * 8 + kw) * C + c
                        msk[t, (dh * 8 + w8) * C + c, 0] = 1.0
    w1t = conv1_w[jnp.asarray(idx.reshape(-1))].reshape(
        len(taps), 32 * C, 32) * jnp.asarray(msk)

    hperm = np.arange(3200, dtype=np.int32)
    hperm[:3136] = hperm[:3136].reshape(7, 7, 64).transpose(1, 0, 2).reshape(-1)
    whp = wh[jnp.asarray(hperm)]

    args = (x, w1t, conv1_b, conv2_w, conv2_b, conv3_w, conv3_b,
            whp, bh, wq, bq)
    in_specs = [
        pl.BlockSpec((1, 8, nb * SB, 12 * 8 * C), lambda i: (i, 0, 0, 0)),
        pl.BlockSpec(w1t.shape, lambda i: (0, 0, 0)),
    ] + [pl.BlockSpec(a.shape, lambda i: (0,) * a.ndim) for a in args[2:]]

    out = pl.pallas_call(
        functools.partial(_fused_kernel, nb=nb),
        out_shape=jax.ShapeDtypeStruct((2, nb, A), jnp.float32),
        grid=(2,),
        in_specs=in_specs,
        out_specs=pl.BlockSpec((1, nb, A), lambda i: (i, 0, 0)),
        compiler_params=pltpu.CompilerParams(
            dimension_semantics=("parallel",)),
    )(*args)
    return out.reshape(B, A)
